# baseline (speedup 1.0000x reference)
import functools

import jax
import jax.numpy as jnp
from jax import lax
from jax.experimental import pallas as pl
from jax.experimental.pallas import tpu as pltpu
from jax.experimental.pallas import tpu_sc as plsc

_DROP_RATIO = 0.75


def _sc_gather(xf, flat_idx, BK, D):
    info = plsc.get_sparse_core_info()
    NC, NS = info.num_cores, info.num_subcores
    NW = NC * NS
    b_per_w = BK // NW
    CH = 64
    n_ch = b_per_w // CH

    idx2d = flat_idx.reshape(BK // CH, CH)
    mesh = plsc.VectorSubcoreMesh(core_axis_name="c", subcore_axis_name="s")

    @functools.partial(
        pl.kernel,
        mesh=mesh,
        out_type=jax.ShapeDtypeStruct((BK, D), jnp.float32),
        scratch_types=[
            pltpu.VMEM((n_ch, CH), jnp.int32),
            pltpu.VMEM((2, CH, D), jnp.float32),
            pltpu.SemaphoreType.DMA,
            pltpu.SemaphoreType.DMA,
            pltpu.SemaphoreType.DMA,
            pltpu.SemaphoreType.DMA,
        ],
    )
    def gather_kernel(x_hbm, idx_hbm, out_hbm, idx_v, bufs, g0, g1, s0, s1):
        wid = lax.axis_index("s") * NC + lax.axis_index("c")
        base = wid * b_per_w
        pltpu.sync_copy(idx_hbm.at[pl.ds(wid * n_ch, n_ch)], idx_v)
        gsem = (g0, g1)
        ssem = (s0, s1)
        d_g = [None, None]
        d_s = [None, None]
        for c in range(min(2, n_ch)):
            d_g[c] = pltpu.async_copy(x_hbm.at[idx_v.at[c]], bufs.at[c], gsem[c])
        for c in range(n_ch):
            b = c % 2
            d_g[b].wait()
            d_s[b] = pltpu.async_copy(
                bufs.at[b], out_hbm.at[pl.ds(base + c * CH, CH)], ssem[b]
            )
            if c + 2 < n_ch:
                d_s[b].wait()
                d_g[b] = pltpu.async_copy(
                    x_hbm.at[idx_v.at[c + 2]], bufs.at[b], gsem[b]
                )
        for c in range(max(0, n_ch - 2), n_ch):
            d_s[c % 2].wait()

    return gather_kernel(xf, idx2d)


def kernel(x):
    B, N, D = x.shape
    keep_ratio = 1.0 - _DROP_RATIO
    num_keep = max(1, int(N * keep_ratio))
    step = N / num_keep
    base_indices = jnp.arange(num_keep, dtype=jnp.float32) * step
    offs_key = jax.random.key(42)
    offsets = jax.random.uniform(offs_key, (B, num_keep), dtype=jnp.float32) * (step * 0.5)
    indices = (base_indices[None, :] + offsets).astype(jnp.int32)
    indices = jnp.clip(indices, 0, N - 1)

    flat_idx = (indices + jnp.arange(B, dtype=jnp.int32)[:, None] * N).reshape(-1)
    xf = x.reshape(B * N, D)
    out = _sc_gather(xf, flat_idx, B * num_keep, D)
    return out.reshape(B, num_keep, D), indices


# speedup vs baseline: 1.0204x; 1.0204x over previous
import functools

import jax
import jax.numpy as jnp
from jax import lax
from jax.experimental import pallas as pl
from jax.experimental.pallas import tpu as pltpu
from jax.experimental.pallas import tpu_sc as plsc

_DROP_RATIO = 0.75


def _sc_gather(xf, flat_idx, BK, D):
    info = plsc.get_sparse_core_info()
    NC, NS = info.num_cores, info.num_subcores
    NW = NC * NS
    b_per_w = BK // NW
    CH = 32
    NBUF = 4
    n_ch = b_per_w // CH

    idx2d = flat_idx.reshape(BK // CH, CH)
    mesh = plsc.VectorSubcoreMesh(core_axis_name="c", subcore_axis_name="s")

    @functools.partial(
        pl.kernel,
        mesh=mesh,
        out_type=jax.ShapeDtypeStruct((BK, D), jnp.float32),
        scratch_types=[
            pltpu.VMEM((n_ch, CH), jnp.int32),
            pltpu.VMEM((NBUF, CH, D), jnp.float32),
        ]
        + [pltpu.SemaphoreType.DMA] * (2 * NBUF),
    )
    def gather_kernel(x_hbm, idx_hbm, out_hbm, idx_v, bufs, *sems):
        gsem, ssem = sems[:NBUF], sems[NBUF:]
        wid = lax.axis_index("s") * NC + lax.axis_index("c")
        base = wid * b_per_w
        pltpu.sync_copy(idx_hbm.at[pl.ds(wid * n_ch, n_ch)], idx_v)
        d_g = [None] * NBUF
        d_s = [None] * NBUF
        for c in range(min(NBUF, n_ch)):
            d_g[c] = pltpu.async_copy(x_hbm.at[idx_v.at[c]], bufs.at[c], gsem[c])
        for c in range(n_ch):
            b = c % NBUF
            d_g[b].wait()
            d_s[b] = pltpu.async_copy(
                bufs.at[b], out_hbm.at[pl.ds(base + c * CH, CH)], ssem[b]
            )
            if c + NBUF < n_ch:
                d_s[b].wait()
                d_g[b] = pltpu.async_copy(
                    x_hbm.at[idx_v.at[c + NBUF]], bufs.at[b], gsem[b]
                )
        for c in range(max(0, n_ch - NBUF), n_ch):
            d_s[c % NBUF].wait()

    return gather_kernel(xf, idx2d)


def kernel(x):
    B, N, D = x.shape
    keep_ratio = 1.0 - _DROP_RATIO
    num_keep = max(1, int(N * keep_ratio))
    step = N / num_keep
    base_indices = jnp.arange(num_keep, dtype=jnp.float32) * step
    offs_key = jax.random.key(42)
    offsets = jax.random.uniform(offs_key, (B, num_keep), dtype=jnp.float32) * (step * 0.5)
    indices = (base_indices[None, :] + offsets).astype(jnp.int32)
    indices = jnp.clip(indices, 0, N - 1)

    flat_idx = (indices + jnp.arange(B, dtype=jnp.int32)[:, None] * N).reshape(-1)
    xf = x.reshape(B * N, D)
    out = _sc_gather(xf, flat_idx, B * num_keep, D)
    return out.reshape(B, num_keep, D), indices
